# baseline (speedup 1.0000x reference)
import functools

import jax
import jax.numpy as jnp
from jax.experimental import pallas as pl
from jax.experimental.pallas import tpu as pltpu


def _vq_kernel(x_ref, w_ref, x2_ref, w2_ref, out_ref, loss_ref, *, k_total):
    x = x_ref[:]
    w = w_ref[:]

    x2 = x2_ref[:]
    w2 = w2_ref[:]
    n2dot = jax.lax.dot_general(
        -2.0 * x, w, (((1,), (1,)), ((), ())),
        preferred_element_type=jnp.float32)
    d2 = x2 + w2 + n2dot
    dist = jnp.sqrt(jnp.maximum(d2, 0.0))

    m = jnp.min(dist, axis=1, keepdims=True)
    iota = jax.lax.broadcasted_iota(jnp.int32, dist.shape, 1)
    idx = jnp.min(jnp.where(dist == m, iota, k_total), axis=1, keepdims=True)
    onehot = (iota == idx).astype(jnp.float32)

    q = jax.lax.dot_general(
        onehot, w, (((1,), (0,)), ((), ())),
        preferred_element_type=jnp.float32)

    out_ref[:] = x + (q - x)

    loss_ref[...] = jnp.sum(m * m).reshape(1, 1, 1)


def kernel(weights_flat, W):
    n, d = weights_flat.shape
    k, _ = W.shape
    bn = 1536
    grid = (n // bn,)

    x2 = jnp.sum(weights_flat * weights_flat, axis=1, keepdims=True)
    w2 = jnp.sum(W * W, axis=1)[None, :]

    out, loss_parts = pl.pallas_call(
        functools.partial(_vq_kernel, k_total=k),
        grid=grid,
        in_specs=[
            pl.BlockSpec((bn, d), lambda i: (i, 0)),
            pl.BlockSpec((k, d), lambda i: (0, 0)),
            pl.BlockSpec((bn, 1), lambda i: (i, 0)),
            pl.BlockSpec((1, k), lambda i: (0, 0)),
        ],
        out_specs=[
            pl.BlockSpec((bn, d), lambda i: (i, 0)),
            pl.BlockSpec((1, 1, 1), lambda i: (i, 0, 0)),
        ],
        out_shape=[
            jax.ShapeDtypeStruct((n, d), jnp.float32),
            jax.ShapeDtypeStruct((n // bn, 1, 1), jnp.float32),
        ],
        compiler_params=pltpu.CompilerParams(
            dimension_semantics=("parallel",)),
    )(weights_flat, W, x2, w2)

    mean_sq = jnp.sum(loss_parts) / (n * d)
    vq_loss = mean_sq + 0.1 * mean_sq
    return (out, vq_loss)


# speedup vs baseline: 1.0118x; 1.0118x over previous
import functools

import jax
import jax.numpy as jnp
from jax.experimental import pallas as pl
from jax.experimental.pallas import tpu as pltpu


def _vq_kernel(x_ref, w_ref, x2_ref, w2_ref, out_ref, loss_ref, *, k_total):
    x = x_ref[:]
    w = w_ref[:]

    x2 = x2_ref[:]
    w2 = w2_ref[:]
    n2dot = jax.lax.dot_general(
        -2.0 * x, w, (((1,), (1,)), ((), ())),
        preferred_element_type=jnp.float32)
    d2 = x2 + w2 + n2dot
    dist = jnp.sqrt(jnp.maximum(d2, 0.0))

    m = jnp.min(dist, axis=1, keepdims=True)
    iota = jax.lax.broadcasted_iota(jnp.int32, dist.shape, 1)
    idx = jnp.min(jnp.where(dist == m, iota, k_total), axis=1, keepdims=True)
    onehot = (iota == idx).astype(jnp.float32)

    q = jax.lax.dot_general(
        onehot, w, (((1,), (0,)), ((), ())),
        preferred_element_type=jnp.float32)

    out_ref[:] = x + (q - x)

    loss_ref[...] = jnp.sum(m * m).reshape(1, 1, 1)


def kernel(weights_flat, W):
    n, d = weights_flat.shape
    k, _ = W.shape
    bn = 1024
    grid = (n // bn,)

    x2 = jnp.sum(weights_flat * weights_flat, axis=1, keepdims=True)
    w2 = jnp.sum(W * W, axis=1)[None, :]

    out, loss_parts = pl.pallas_call(
        functools.partial(_vq_kernel, k_total=k),
        grid=grid,
        in_specs=[
            pl.BlockSpec((bn, d), lambda i: (i, 0)),
            pl.BlockSpec((k, d), lambda i: (0, 0)),
            pl.BlockSpec((bn, 1), lambda i: (i, 0)),
            pl.BlockSpec((1, k), lambda i: (0, 0)),
        ],
        out_specs=[
            pl.BlockSpec((bn, d), lambda i: (i, 0)),
            pl.BlockSpec((1, 1, 1), lambda i: (i, 0, 0)),
        ],
        out_shape=[
            jax.ShapeDtypeStruct((n, d), jnp.float32),
            jax.ShapeDtypeStruct((n // bn, 1, 1), jnp.float32),
        ],
    )(weights_flat, W, x2, w2)

    mean_sq = jnp.sum(loss_parts) / (n * d)
    vq_loss = mean_sq + 0.1 * mean_sq
    return (out, vq_loss)
